# Initial kernel scaffold; baseline (speedup 1.0000x reference)
import jax
import jax.numpy as jnp
from jax.experimental import pallas as pl


def kernel(x, edge_index, batch, W, b):
    raise NotImplementedError("write your pallas kernel here")


# speedup vs baseline: 200.0361x; 200.0361x over previous
import functools

import jax
import jax.numpy as jnp
from jax import lax
from jax.experimental import pallas as pl
from jax.experimental.pallas import tpu as pltpu
from jax.experimental.pallas import tpu_sc as plsc

N = 100000
E = 6400000
G = 1024

NC = 2
NS = 16
NW = NC * NS
L = 16

RW = 128
KR = 16
ECH = KR * RW
NECH = E // ECH
EROWS = E // RW

NODE_CH = 1024
NPAD = 100352
NNCH = NPAD // NODE_CH
DUMP = NPAD // NS

GP = 1056

_mesh = plsc.VectorSubcoreMesh(
    core_axis_name="c", subcore_axis_name="s", num_cores=NC, num_subcores=NS)
f32 = jnp.float32
i32 = jnp.int32


def _rsqrt(d):
    i = lax.bitcast_convert_type(d, i32)
    i = jnp.int32(0x5F3759DF) - lax.shift_right_logical(i, 1)
    y = lax.bitcast_convert_type(i, f32)
    for _ in range(3):
        y = y * (jnp.float32(1.5) - jnp.float32(0.5) * d * y * y)
    return y


def _zero_vmem(ref, n):
    for v in range(n // L):
        ref[pl.ds(v * L, L)] = jnp.zeros((L,), f32)


def _zero_shared(shared, zbuf, sid):
    nz = shared.shape[0] // NODE_CH
    for it in range((nz + NS - 1) // NS):
        c = sid + NS * it

        @pl.when(c < nz)
        def _():
            pltpu.sync_copy(zbuf, shared.at[pl.ds(c * NODE_CH, NODE_CH)])


def _dump_shared(shared, dbuf, hbm, base, sid):
    off = sid * DUMP
    pltpu.sync_copy(shared.at[pl.ds(off, DUMP)], dbuf)
    pltpu.sync_copy(dbuf, hbm.at[pl.ds(base + off, DUMP)])


@functools.partial(
    pl.kernel,
    out_type=jax.ShapeDtypeStruct((NC * NPAD,), f32),
    mesh=_mesh,
    compiler_params=pltpu.CompilerParams(needs_layout_passes=False),
    scratch_types=[
        pltpu.VMEM((KR, RW), i32),
        pltpu.VMEM((RW,), f32),
        pltpu.VMEM((NODE_CH,), f32),
        pltpu.VMEM((DUMP,), f32),
        pltpu.VMEM_SHARED((NPAD,), f32),
    ],
)
def _k1(dst_hbm, degp_hbm, ibuf, ones_v, zbuf, dbuf, shared):
    cid = lax.axis_index("c")
    sid = lax.axis_index("s")
    wid = sid * NC + cid

    _zero_vmem(zbuf, NODE_CH)
    for v in range(RW // L):
        ones_v[pl.ds(v * L, L)] = jnp.ones((L,), f32)
    _zero_shared(shared, zbuf, sid)
    plsc.subcore_barrier()

    def body(it, carry):
        c = wid + NW * it

        @pl.when(c < NECH)
        def _():
            pltpu.sync_copy(dst_hbm.at[pl.ds(c * KR, KR)], ibuf)
            for j in range(KR):
                pltpu.sync_copy(ones_v, shared.at[ibuf.at[j]], add=True)

        return carry

    lax.fori_loop(0, (NECH + NW - 1) // NW, body, 0)
    plsc.subcore_barrier()
    _dump_shared(shared, dbuf, degp_hbm, cid * NPAD, sid)


@functools.partial(
    pl.kernel,
    out_type=(
        jax.ShapeDtypeStruct((NPAD,), f32),
        jax.ShapeDtypeStruct((NPAD,), f32),
        jax.ShapeDtypeStruct((NPAD,), f32),
    ),
    mesh=_mesh,
    compiler_params=pltpu.CompilerParams(needs_layout_passes=False),
    scratch_types=[
        pltpu.VMEM((NODE_CH,), f32),
        pltpu.VMEM((NODE_CH,), f32),
        pltpu.VMEM((NODE_CH,), f32),
        pltpu.VMEM((NODE_CH,), f32),
        pltpu.VMEM((NODE_CH,), f32),
        pltpu.VMEM((NODE_CH,), f32),
        pltpu.VMEM((16,), f32),
        pltpu.VMEM((NODE_CH,), f32),
        pltpu.VMEM((NODE_CH,), f32),
        pltpu.VMEM((NODE_CH,), f32),
    ],
)
def _k2(x0h, x1h, x2h, x3h, degp_h, wb_h, z_h, n_h, p_h,
        x0b, x1b, x2b, x3b, d0b, d1b, wbuf, zb, nb, pb):
    cid = lax.axis_index("c")
    sid = lax.axis_index("s")
    wid = sid * NC + cid

    pltpu.sync_copy(wb_h, wbuf)
    wv = wbuf[pl.ds(0, L)]
    w0 = wv[0]
    w1 = wv[1]
    w2 = wv[2]
    w3 = wv[3]

    def body(it, carry):
        c = wid + NW * it

        @pl.when(c < NNCH)
        def _():
            base = c * NODE_CH
            pltpu.sync_copy(x0h.at[pl.ds(base, NODE_CH)], x0b)
            pltpu.sync_copy(x1h.at[pl.ds(base, NODE_CH)], x1b)
            pltpu.sync_copy(x2h.at[pl.ds(base, NODE_CH)], x2b)
            pltpu.sync_copy(x3h.at[pl.ds(base, NODE_CH)], x3b)
            pltpu.sync_copy(degp_h.at[pl.ds(base, NODE_CH)], d0b)
            pltpu.sync_copy(degp_h.at[pl.ds(NPAD + base, NODE_CH)], d1b)

            def inner(v, carry2):
                sl = pl.ds(v * L, L)
                p = (x0b[sl] * w0 + x1b[sl] * w1
                     + x2b[sl] * w2 + x3b[sl] * w3)
                d = d0b[sl] + d1b[sl] + jnp.float32(1.0)
                d = jnp.maximum(d, jnp.float32(1.0))
                y = _rsqrt(d)
                nb[sl] = y
                pb[sl] = p
                zb[sl] = p * y
                return carry2

            lax.fori_loop(0, NODE_CH // L, inner, 0)
            pltpu.sync_copy(zb, z_h.at[pl.ds(base, NODE_CH)])
            pltpu.sync_copy(nb, n_h.at[pl.ds(base, NODE_CH)])
            pltpu.sync_copy(pb, p_h.at[pl.ds(base, NODE_CH)])

        return carry

    lax.fori_loop(0, (NNCH + NW - 1) // NW, body, 0)


@functools.partial(
    pl.kernel,
    out_type=jax.ShapeDtypeStruct((NC * NPAD,), f32),
    mesh=_mesh,
    compiler_params=pltpu.CompilerParams(needs_layout_passes=False),
    scratch_types=[
        pltpu.VMEM((NPAD,), f32),
        pltpu.VMEM((KR, RW), i32),
        pltpu.VMEM((KR, RW), i32),
        pltpu.VMEM((KR, RW), f32),
        pltpu.VMEM((NODE_CH,), f32),
        pltpu.VMEM((DUMP,), f32),
        pltpu.VMEM_SHARED((NPAD,), f32),
    ],
)
def _k3(src_hbm, dst_hbm, z_hbm, accp_hbm,
        ztab, sbuf, dbuf, vbuf, zbuf, dmpb, shared):
    cid = lax.axis_index("c")
    sid = lax.axis_index("s")
    wid = sid * NC + cid

    _zero_vmem(zbuf, NODE_CH)
    _zero_shared(shared, zbuf, sid)
    pltpu.sync_copy(z_hbm, ztab)
    plsc.subcore_barrier()

    def body(it, carry):
        c = wid + NW * it

        @pl.when(c < NECH)
        def _():
            pltpu.sync_copy(src_hbm.at[pl.ds(c * KR, KR)], sbuf)
            pltpu.sync_copy(dst_hbm.at[pl.ds(c * KR, KR)], dbuf)
            for j in range(KR):
                for v in range(RW // L):
                    idx = sbuf[j, pl.ds(v * L, L)]
                    vbuf[j, pl.ds(v * L, L)] = plsc.load_gather(ztab, [idx])
                pltpu.sync_copy(vbuf.at[j], shared.at[dbuf.at[j]], add=True)

        return carry

    lax.fori_loop(0, (NECH + NW - 1) // NW, body, 0)
    plsc.subcore_barrier()
    _dump_shared(shared, dmpb, accp_hbm, cid * NPAD, sid)


@functools.partial(
    pl.kernel,
    out_type=(
        jax.ShapeDtypeStruct((NC * GP,), f32),
        jax.ShapeDtypeStruct((NC * GP,), f32),
    ),
    mesh=_mesh,
    compiler_params=pltpu.CompilerParams(needs_layout_passes=False),
    scratch_types=[
        pltpu.VMEM((NODE_CH,), f32),
        pltpu.VMEM((NODE_CH,), f32),
        pltpu.VMEM((NODE_CH,), f32),
        pltpu.VMEM((NODE_CH,), f32),
        pltpu.VMEM((NODE_CH // RW, RW), i32),
        pltpu.VMEM((NODE_CH // RW, RW), f32),
        pltpu.VMEM((RW,), f32),
        pltpu.VMEM((16,), f32),
        pltpu.VMEM((GP,), f32),
        pltpu.VMEM_SHARED((GP,), f32),
        pltpu.VMEM_SHARED((GP,), f32),
    ],
)
def _k4(accp_h, nrm_h, p_h, batch_h, wb_h, sump_h, cntp_h,
        a0b, a1b, nb, pb, bbuf, obuf, ones_v, wbuf, gbuf, shS, shC):
    cid = lax.axis_index("c")
    sid = lax.axis_index("s")
    wid = sid * NC + cid
    nrow = NODE_CH // RW

    pltpu.sync_copy(wb_h, wbuf)
    bconst = wbuf[pl.ds(0, L)][4]
    _zero_vmem(gbuf, GP)
    for v in range(RW // L):
        ones_v[pl.ds(v * L, L)] = jnp.ones((L,), f32)

    @pl.when(sid == 0)
    def _():
        pltpu.sync_copy(gbuf, shS)
        pltpu.sync_copy(gbuf, shC)

    plsc.subcore_barrier()

    def body(it, carry):
        c = wid + NW * it

        @pl.when(c < NNCH)
        def _():
            base = c * NODE_CH
            pltpu.sync_copy(accp_h.at[pl.ds(base, NODE_CH)], a0b)
            pltpu.sync_copy(accp_h.at[pl.ds(NPAD + base, NODE_CH)], a1b)
            pltpu.sync_copy(nrm_h.at[pl.ds(base, NODE_CH)], nb)
            pltpu.sync_copy(p_h.at[pl.ds(base, NODE_CH)], pb)
            pltpu.sync_copy(batch_h.at[pl.ds(c * nrow, nrow)], bbuf)
            for j in range(nrow):
                for v in range(RW // L):
                    sl = pl.ds(j * RW + v * L, L)
                    y = nb[sl]
                    o = y * (a0b[sl] + a1b[sl]) + pb[sl] * y * y + bconst
                    obuf[j, pl.ds(v * L, L)] = o
                pltpu.sync_copy(obuf.at[j], shS.at[bbuf.at[j]], add=True)
                pltpu.sync_copy(ones_v, shC.at[bbuf.at[j]], add=True)

        return carry

    lax.fori_loop(0, (NNCH + NW - 1) // NW, body, 0)
    plsc.subcore_barrier()

    @pl.when(sid == 0)
    def _():
        pltpu.sync_copy(shS, gbuf)
        pltpu.sync_copy(gbuf, sump_h.at[pl.ds(cid * GP, GP)])
        pltpu.sync_copy(shC, gbuf)
        pltpu.sync_copy(gbuf, cntp_h.at[pl.ds(cid * GP, GP)])


@functools.partial(
    pl.kernel,
    out_type=jax.ShapeDtypeStruct((G,), f32),
    mesh=_mesh,
    compiler_params=pltpu.CompilerParams(needs_layout_passes=False),
    scratch_types=[
        pltpu.VMEM((NC * GP,), f32),
        pltpu.VMEM((NC * GP,), f32),
        pltpu.VMEM((2 * L,), f32),
    ],
)
def _k5(sump_h, cntp_h, logits_h, sbuf, cbuf, obuf):
    cid = lax.axis_index("c")
    sid = lax.axis_index("s")
    wid = sid * NC + cid
    per_w = G // NW

    pltpu.sync_copy(sump_h, sbuf)
    pltpu.sync_copy(cntp_h, cbuf)
    base = wid * per_w
    for v in range(per_w // L):
        sl = pl.ds(base + v * L, L)
        slp = pl.ds(GP + base + v * L, L)
        s = sbuf[sl] + sbuf[slp]
        cnt = cbuf[sl] + cbuf[slp]
        obuf[pl.ds(v * L, L)] = s / jnp.maximum(cnt, jnp.float32(1.0))
    pltpu.sync_copy(obuf, logits_h.at[pl.ds(base, per_w)])


def kernel(x, edge_index, batch, W, b):
    src2 = edge_index[0].reshape(EROWS, RW)
    dst2 = edge_index[1].reshape(EROWS, RW)
    xp = jnp.pad(x, ((0, NPAD - N), (0, 0)))
    x0, x1, x2, x3 = (xp[:, j] for j in range(4))
    batchp = jnp.pad(batch, (0, NPAD - N), constant_values=G).reshape(
        NPAD // RW, RW)
    wb = jnp.zeros((16,), f32).at[:4].set(W[:, 0]).at[4].set(b[0])

    degp = _k1(dst2)
    z, nrm, p = _k2(x0, x1, x2, x3, degp, wb)
    accp = _k3(src2, dst2, z)
    sump, cntp = _k4(accp, nrm, p, batchp, wb)
    return _k5(sump, cntp)


# speedup vs baseline: 431.3670x; 2.1564x over previous
import functools

import jax
import jax.numpy as jnp
from jax import lax
from jax.experimental import pallas as pl
from jax.experimental.pallas import tpu as pltpu
from jax.experimental.pallas import tpu_sc as plsc

N = 100000
E = 6400000
G = 1024

NC = 2
NS = 16
NW = NC * NS
L = 16

RW = 128
KR = 16
ECH = KR * RW
NECH = E // ECH
EROWS = E // RW

NODE_CH = 1024
NPAD = 100352
NNCH = NPAD // NODE_CH
DUMP = NPAD // NS
BLK = NW * NODE_CH

GP = 1056

_mesh = plsc.VectorSubcoreMesh(
    core_axis_name="c", subcore_axis_name="s", num_cores=NC, num_subcores=NS)
_params = pltpu.CompilerParams(needs_layout_passes=False)
f32 = jnp.float32
i32 = jnp.int32


def _rsqrt(d):
    i = lax.bitcast_convert_type(d, i32)
    i = jnp.int32(0x5F3759DF) - lax.shift_right_logical(i, 1)
    y = lax.bitcast_convert_type(i, f32)
    for _ in range(3):
        y = y * (jnp.float32(1.5) - jnp.float32(0.5) * d * y * y)
    return y


def _zero_vmem(ref, n):
    for v in range(n // L):
        ref[pl.ds(v * L, L)] = jnp.zeros((L,), f32)


def _zero_vmem_big(ref, n):
    def body(it, carry):
        base = it * 256
        for k in range(16):
            ref[pl.ds(base + k * L, L)] = jnp.zeros((L,), f32)
        return carry

    lax.fori_loop(0, n // 256, body, 0)


def _zero_shared(shared, zbuf, sid):
    nz = shared.shape[0] // NODE_CH
    for it in range((nz + NS - 1) // NS):
        c = sid + NS * it

        @pl.when(c < nz)
        def _():
            pltpu.sync_copy(zbuf, shared.at[pl.ds(c * NODE_CH, NODE_CH)])


def _dump_shared(shared, dbuf, hbm, base, sid):
    half = DUMP // 2
    for k in range(2):
        off = sid * DUMP + k * half
        pltpu.sync_copy(shared.at[pl.ds(off, half)], dbuf)
        pltpu.sync_copy(dbuf, hbm.at[pl.ds(base + off, half)])


@functools.partial(
    pl.kernel,
    out_type=jax.ShapeDtypeStruct((NNCH * BLK,), f32),
    mesh=_mesh,
    compiler_params=_params,
    scratch_types=[
        pltpu.VMEM((NPAD,), f32),
        pltpu.VMEM((2, KR, RW), i32),
        pltpu.SemaphoreType.DMA,
        pltpu.SemaphoreType.DMA,
        pltpu.SemaphoreType.DMA,
    ],
)
def _k1(dst_hbm, degp_hbm, acc, ibuf, sem0, sem1, semd):
    cid = lax.axis_index("c")
    sid = lax.axis_index("s")
    wid = sid * NC + cid
    sems = (sem0, sem1)
    ones = jnp.ones((L,), f32)

    _zero_vmem_big(acc, NPAD)

    def start_in(i, slot):
        c = wid + NW * i

        @pl.when(c < NECH)
        def _():
            pltpu.async_copy(dst_hbm.at[pl.ds(c * KR, KR)], ibuf.at[slot],
                             sems[slot])

    def wait_in(slot):
        pltpu.make_async_copy(dst_hbm.at[pl.ds(0, KR)], ibuf.at[slot],
                              sems[slot]).wait()

    start_in(0, 0)

    def body(it, carry):
        for b in range(2):
            i = 2 * it + b
            c = wid + NW * i
            start_in(i + 1, 1 - b)

            @pl.when(c < NECH)
            def _():
                wait_in(b)
                for j in range(KR):
                    for g in range(RW // L):
                        idx = ibuf[b, j, pl.ds(g * L, L)]
                        plsc.addupdate_scatter(acc, [idx], ones)

        return carry

    lax.fori_loop(0, (NECH + NW - 1) // NW // 2, body, 0)

    for cnk in range(NNCH):
        off = cnk * BLK + wid * NODE_CH
        pltpu.async_copy(acc.at[pl.ds(cnk * NODE_CH, NODE_CH)],
                         degp_hbm.at[pl.ds(off, NODE_CH)], semd)
    for cnk in range(NNCH):
        pltpu.make_async_copy(acc.at[pl.ds(0, NODE_CH)],
                              degp_hbm.at[pl.ds(0, NODE_CH)], semd).wait()


@functools.partial(
    pl.kernel,
    out_type=(
        jax.ShapeDtypeStruct((NPAD,), f32),
        jax.ShapeDtypeStruct((NPAD,), f32),
        jax.ShapeDtypeStruct((NPAD,), f32),
    ),
    mesh=_mesh,
    compiler_params=_params,
    scratch_types=[
        pltpu.VMEM((NODE_CH,), f32),
        pltpu.VMEM((NODE_CH,), f32),
        pltpu.VMEM((NODE_CH,), f32),
        pltpu.VMEM((NODE_CH,), f32),
        pltpu.VMEM((BLK,), f32),
        pltpu.VMEM((16,), f32),
        pltpu.VMEM((NODE_CH,), f32),
        pltpu.VMEM((NODE_CH,), f32),
        pltpu.VMEM((NODE_CH,), f32),
        pltpu.SemaphoreType.DMA,
    ],
)
def _k2(x0h, x1h, x2h, x3h, degp_h, wb_h, z_h, n_h, p_h,
        x0b, x1b, x2b, x3b, dpb, wbuf, zb, nb, pb, semi):
    cid = lax.axis_index("c")
    sid = lax.axis_index("s")
    wid = sid * NC + cid

    pltpu.sync_copy(wb_h, wbuf)
    wv = wbuf[pl.ds(0, L)]
    w0 = wv[0]
    w1 = wv[1]
    w2 = wv[2]
    w3 = wv[3]

    def body(it, carry):
        c = wid + NW * it

        @pl.when(c < NNCH)
        def _():
            base = c * NODE_CH
            pltpu.async_copy(x0h.at[pl.ds(base, NODE_CH)], x0b, semi)
            pltpu.async_copy(x1h.at[pl.ds(base, NODE_CH)], x1b, semi)
            pltpu.async_copy(x2h.at[pl.ds(base, NODE_CH)], x2b, semi)
            pltpu.async_copy(x3h.at[pl.ds(base, NODE_CH)], x3b, semi)
            pltpu.async_copy(degp_h.at[pl.ds(c * BLK, BLK)], dpb, semi)
            for _ in range(4):
                pltpu.make_async_copy(x0h.at[pl.ds(0, NODE_CH)], x0b,
                                      semi).wait()
            pltpu.make_async_copy(degp_h.at[pl.ds(0, BLK)], dpb, semi).wait()

            def inner(v, carry2):
                sl = pl.ds(v * L, L)
                p = (x0b[sl] * w0 + x1b[sl] * w1
                     + x2b[sl] * w2 + x3b[sl] * w3)
                acc = jnp.zeros((L,), f32)
                for w in range(NW):
                    acc = acc + dpb[pl.ds(w * NODE_CH + v * L, L)]
                d = acc + jnp.float32(1.0)
                d = jnp.maximum(d, jnp.float32(1.0))
                y = _rsqrt(d)
                nb[sl] = y
                pb[sl] = p
                zb[sl] = p * y
                return carry2

            lax.fori_loop(0, NODE_CH // L, inner, 0)
            pltpu.sync_copy(zb, z_h.at[pl.ds(base, NODE_CH)])
            pltpu.sync_copy(nb, n_h.at[pl.ds(base, NODE_CH)])
            pltpu.sync_copy(pb, p_h.at[pl.ds(base, NODE_CH)])

        return carry

    lax.fori_loop(0, (NNCH + NW - 1) // NW, body, 0)


@functools.partial(
    pl.kernel,
    out_type=jax.ShapeDtypeStruct((NC * NPAD,), f32),
    mesh=_mesh,
    compiler_params=_params,
    scratch_types=[
        pltpu.VMEM((NPAD,), f32),
        pltpu.VMEM((2, KR, RW), i32),
        pltpu.VMEM((4, KR, RW), i32),
        pltpu.VMEM((4, KR, RW), f32),
        pltpu.VMEM((DUMP // 2,), f32),
        pltpu.VMEM_SHARED((NPAD,), f32),
        pltpu.SemaphoreType.DMA,
        pltpu.SemaphoreType.DMA,
        pltpu.SemaphoreType.DMA,
        pltpu.SemaphoreType.DMA,
        pltpu.SemaphoreType.DMA,
        pltpu.SemaphoreType.DMA,
    ],
)
def _k3(src_hbm, dst_hbm, z_hbm, accp_hbm,
        ztab, sbuf, dbuf, vbuf, dmpb, shared,
        semi0, semi1, sems0, sems1, sems2, sems3):
    cid = lax.axis_index("c")
    sid = lax.axis_index("s")
    wid = sid * NC + cid
    semi = (semi0, semi1)
    sems = (sems0, sems1, sems2, sems3)

    _zero_vmem(dmpb, NODE_CH)
    _zero_shared(shared, dmpb.at[pl.ds(0, NODE_CH)], sid)
    pltpu.sync_copy(z_hbm, ztab)
    plsc.subcore_barrier()

    def start_in(i, s2, s4):
        c = wid + NW * i

        @pl.when(c < NECH)
        def _():
            pltpu.async_copy(src_hbm.at[pl.ds(c * KR, KR)], sbuf.at[s2],
                             semi[s2])
            pltpu.async_copy(dst_hbm.at[pl.ds(c * KR, KR)], dbuf.at[s4],
                             semi[s2])

    def wait_in(s2, s4):
        pltpu.make_async_copy(src_hbm.at[pl.ds(0, KR)], sbuf.at[s2],
                              semi[s2]).wait()
        pltpu.make_async_copy(dst_hbm.at[pl.ds(0, KR)], dbuf.at[s4],
                              semi[s2]).wait()

    def drain_scatter(s4):
        for j in range(KR):
            pltpu.make_async_copy(z_hbm.at[pl.ds(0, RW)], vbuf.at[s4, j],
                                  sems[s4]).wait()

    start_in(0, 0, 0)

    def body(it, carry):
        for b in range(4):
            i = 4 * it + b
            c = wid + NW * i
            start_in(i + 1, (b + 1) % 2, (b + 1) % 4)

            @pl.when(c < NECH)
            def _():
                wait_in(b % 2, b)

                @pl.when(i >= 2)
                def _():
                    drain_scatter((b + 2) % 4)

                for j in range(KR):
                    for g in range(RW // L):
                        idx = sbuf[b % 2, j, pl.ds(g * L, L)]
                        vbuf[b, j, pl.ds(g * L, L)] = plsc.load_gather(
                            ztab, [idx])
                for j in range(KR):
                    pltpu.async_copy(vbuf.at[b, j],
                                     shared.at[dbuf.at[b, j]],
                                     sems[b], add=True)

        return carry

    nit = (NECH + NW - 1) // NW
    lax.fori_loop(0, (nit + 3) // 4, body, 0)
    i_last = (NECH - 1 - wid) // NW
    for s in range(4):
        @pl.when(jnp.logical_or(i_last % 4 == s, (i_last - 1) % 4 == s))
        def _():
            drain_scatter(s)

    plsc.subcore_barrier()
    _dump_shared(shared, dmpb, accp_hbm, cid * NPAD, sid)


@functools.partial(
    pl.kernel,
    out_type=(
        jax.ShapeDtypeStruct((NW * GP,), f32),
        jax.ShapeDtypeStruct((NW * GP,), f32),
    ),
    mesh=_mesh,
    compiler_params=_params,
    scratch_types=[
        pltpu.VMEM((NODE_CH,), f32),
        pltpu.VMEM((NODE_CH,), f32),
        pltpu.VMEM((NODE_CH,), f32),
        pltpu.VMEM((NODE_CH,), f32),
        pltpu.VMEM((NODE_CH,), i32),
        pltpu.VMEM((16,), f32),
        pltpu.VMEM((GP,), f32),
        pltpu.VMEM((GP,), f32),
        pltpu.SemaphoreType.DMA,
    ],
)
def _k4(accp_h, nrm_h, p_h, batch_h, wb_h, sump_h, cntp_h,
        a0b, a1b, nb, pb, bbuf, wbuf, sumb, cntb, semi):
    cid = lax.axis_index("c")
    sid = lax.axis_index("s")
    wid = sid * NC + cid
    ones = jnp.ones((L,), f32)

    pltpu.sync_copy(wb_h, wbuf)
    bconst = wbuf[pl.ds(0, L)][4]
    _zero_vmem(sumb, GP)
    _zero_vmem(cntb, GP)

    def body(it, carry):
        c = wid + NW * it

        @pl.when(c < NNCH)
        def _():
            base = c * NODE_CH
            pltpu.async_copy(accp_h.at[pl.ds(base, NODE_CH)], a0b, semi)
            pltpu.async_copy(accp_h.at[pl.ds(NPAD + base, NODE_CH)], a1b,
                             semi)
            pltpu.async_copy(nrm_h.at[pl.ds(base, NODE_CH)], nb, semi)
            pltpu.async_copy(p_h.at[pl.ds(base, NODE_CH)], pb, semi)
            pltpu.async_copy(batch_h.at[pl.ds(base, NODE_CH)], bbuf, semi)
            for _ in range(4):
                pltpu.make_async_copy(accp_h.at[pl.ds(0, NODE_CH)], a0b,
                                      semi).wait()
            pltpu.make_async_copy(batch_h.at[pl.ds(0, NODE_CH)], bbuf,
                                  semi).wait()

            def inner(v, carry2):
                sl = pl.ds(v * L, L)
                y = nb[sl]
                o = y * (a0b[sl] + a1b[sl]) + pb[sl] * y * y + bconst
                bi = bbuf[sl]
                plsc.addupdate_scatter(sumb, [bi], o)
                plsc.addupdate_scatter(cntb, [bi], ones)
                return carry2

            lax.fori_loop(0, NODE_CH // L, inner, 0)

        return carry

    lax.fori_loop(0, (NNCH + NW - 1) // NW, body, 0)
    pltpu.sync_copy(sumb, sump_h.at[pl.ds(wid * GP, GP)])
    pltpu.sync_copy(cntb, cntp_h.at[pl.ds(wid * GP, GP)])


@functools.partial(
    pl.kernel,
    out_type=jax.ShapeDtypeStruct((G,), f32),
    mesh=_mesh,
    compiler_params=_params,
    scratch_types=[
        pltpu.VMEM((NW * GP,), f32),
        pltpu.VMEM((NW * GP,), f32),
        pltpu.VMEM((2 * L,), f32),
    ],
)
def _k5(sump_h, cntp_h, logits_h, sbuf, cbuf, obuf):
    cid = lax.axis_index("c")
    sid = lax.axis_index("s")
    wid = sid * NC + cid
    per_w = G // NW

    pltpu.sync_copy(sump_h, sbuf)
    pltpu.sync_copy(cntp_h, cbuf)
    base = wid * per_w
    for v in range(per_w // L):
        s = jnp.zeros((L,), f32)
        cnt = jnp.zeros((L,), f32)
        for w in range(NW):
            s = s + sbuf[pl.ds(w * GP + base + v * L, L)]
            cnt = cnt + cbuf[pl.ds(w * GP + base + v * L, L)]
        obuf[pl.ds(v * L, L)] = s / jnp.maximum(cnt, jnp.float32(1.0))
    pltpu.sync_copy(obuf, logits_h.at[pl.ds(base, per_w)])


def kernel(x, edge_index, batch, W, b):
    src2 = edge_index[0].reshape(EROWS, RW)
    dst2 = edge_index[1].reshape(EROWS, RW)
    xp = jnp.pad(x, ((0, NPAD - N), (0, 0)))
    x0, x1, x2, x3 = (xp[:, j] for j in range(4))
    batchp = jnp.pad(batch, (0, NPAD - N), constant_values=G)
    wb = jnp.zeros((16,), f32).at[:4].set(W[:, 0]).at[4].set(b[0])

    degp = _k1(dst2)
    z, nrm, p = _k2(x0, x1, x2, x3, degp, wb)
    accp = _k3(src2, dst2, z)
    sump, cntp = _k4(accp, nrm, p, batchp, wb)
    return _k5(sump, cntp)


# speedup vs baseline: 493.2876x; 1.1435x over previous
import functools

import jax
import jax.numpy as jnp
from jax import lax
from jax.experimental import pallas as pl
from jax.experimental.pallas import tpu as pltpu
from jax.experimental.pallas import tpu_sc as plsc

N = 100000
E = 6400000
G = 1024

NC = 2
NS = 16
NW = NC * NS
L = 16

RW = 128
KR = 16
ECH = KR * RW
NECH = E // ECH
EROWS = E // RW

NODE_CH = 1024
NPAD = 100352
NNCH = NPAD // NODE_CH
DUMP = NPAD // NS
BLK = NW * NODE_CH

GP = 1056

_mesh = plsc.VectorSubcoreMesh(
    core_axis_name="c", subcore_axis_name="s", num_cores=NC, num_subcores=NS)
_params = pltpu.CompilerParams(needs_layout_passes=False)
f32 = jnp.float32
i32 = jnp.int32


def _rsqrt(d):
    i = lax.bitcast_convert_type(d, i32)
    i = jnp.int32(0x5F3759DF) - lax.shift_right_logical(i, 1)
    y = lax.bitcast_convert_type(i, f32)
    for _ in range(3):
        y = y * (jnp.float32(1.5) - jnp.float32(0.5) * d * y * y)
    return y


def _zero_vmem(ref, n):
    for v in range(n // L):
        ref[pl.ds(v * L, L)] = jnp.zeros((L,), f32)


def _zero_vmem_big(ref, n):
    def body(it, carry):
        base = it * 256
        for k in range(16):
            ref[pl.ds(base + k * L, L)] = jnp.zeros((L,), f32)
        return carry

    lax.fori_loop(0, n // 256, body, 0)


def _zero_shared(shared, zbuf, sid):
    nz = shared.shape[0] // NODE_CH
    for it in range((nz + NS - 1) // NS):
        c = sid + NS * it

        @pl.when(c < nz)
        def _():
            pltpu.sync_copy(zbuf, shared.at[pl.ds(c * NODE_CH, NODE_CH)])


def _dump_shared(shared, dbuf, hbm, base, sid):
    half = DUMP // 2
    for k in range(2):
        off = sid * DUMP + k * half
        pltpu.sync_copy(shared.at[pl.ds(off, half)], dbuf)
        pltpu.sync_copy(dbuf, hbm.at[pl.ds(base + off, half)])


@functools.partial(
    pl.kernel,
    out_type=jax.ShapeDtypeStruct((NC * NPAD,), f32),
    mesh=_mesh,
    compiler_params=_params,
    scratch_types=[
        pltpu.VMEM((4, KR, RW), i32),
        pltpu.VMEM((RW,), f32),
        pltpu.VMEM((DUMP // 2,), f32),
        pltpu.VMEM_SHARED((NPAD,), f32),
        pltpu.SemaphoreType.DMA,
        pltpu.SemaphoreType.DMA,
        pltpu.SemaphoreType.DMA,
        pltpu.SemaphoreType.DMA,
        pltpu.SemaphoreType.DMA,
        pltpu.SemaphoreType.DMA,
    ],
)
def _k1(dst_hbm, degp_hbm, dbuf, ones_v, dmpb, shared,
        semi0, semi1, sems0, sems1, sems2, sems3):
    cid = lax.axis_index("c")
    sid = lax.axis_index("s")
    wid = sid * NC + cid
    semi = (semi0, semi1)
    sems = (sems0, sems1, sems2, sems3)

    _zero_vmem(dmpb, NODE_CH)
    for v in range(RW // L):
        ones_v[pl.ds(v * L, L)] = jnp.ones((L,), f32)
    _zero_shared(shared, dmpb.at[pl.ds(0, NODE_CH)], sid)
    plsc.subcore_barrier()

    def start_in(i, s4):
        c = wid + NW * i

        @pl.when(c < NECH)
        def _():
            pltpu.async_copy(dst_hbm.at[pl.ds(c * KR, KR)], dbuf.at[s4],
                             semi[s4 % 2])

    def wait_in(s4):
        pltpu.make_async_copy(dst_hbm.at[pl.ds(0, KR)], dbuf.at[s4],
                              semi[s4 % 2]).wait()

    def drain_scatter(s4):
        for j in range(KR):
            pltpu.make_async_copy(degp_hbm.at[pl.ds(0, RW)], ones_v,
                                  sems[s4]).wait()

    start_in(0, 0)

    def body(it, carry):
        for b in range(4):
            i = 4 * it + b
            c = wid + NW * i
            start_in(i + 1, (b + 1) % 4)

            @pl.when(c < NECH)
            def _():
                wait_in(b)

                @pl.when(i >= 2)
                def _():
                    drain_scatter((b + 2) % 4)

                for j in range(KR):
                    pltpu.async_copy(ones_v, shared.at[dbuf.at[b, j]],
                                     sems[b], add=True)

        return carry

    nit = (NECH + NW - 1) // NW
    lax.fori_loop(0, (nit + 3) // 4, body, 0)
    i_last = (NECH - 1 - wid) // NW
    for s in range(4):
        @pl.when(jnp.logical_or(i_last % 4 == s, (i_last - 1) % 4 == s))
        def _():
            drain_scatter(s)

    plsc.subcore_barrier()
    _dump_shared(shared, dmpb, degp_hbm, cid * NPAD, sid)


@functools.partial(
    pl.kernel,
    out_type=(
        jax.ShapeDtypeStruct((NPAD,), f32),
        jax.ShapeDtypeStruct((NPAD,), f32),
        jax.ShapeDtypeStruct((NPAD,), f32),
    ),
    mesh=_mesh,
    compiler_params=_params,
    scratch_types=[
        pltpu.VMEM((NODE_CH,), f32),
        pltpu.VMEM((NODE_CH,), f32),
        pltpu.VMEM((NODE_CH,), f32),
        pltpu.VMEM((NODE_CH,), f32),
        pltpu.VMEM((NODE_CH,), f32),
        pltpu.VMEM((NODE_CH,), f32),
        pltpu.VMEM((16,), f32),
        pltpu.VMEM((NODE_CH,), f32),
        pltpu.VMEM((NODE_CH,), f32),
        pltpu.VMEM((NODE_CH,), f32),
        pltpu.SemaphoreType.DMA,
    ],
)
def _k2(x0h, x1h, x2h, x3h, degp_h, wb_h, z_h, n_h, p_h,
        x0b, x1b, x2b, x3b, d0b, d1b, wbuf, zb, nb, pb, semi):
    cid = lax.axis_index("c")
    sid = lax.axis_index("s")
    wid = sid * NC + cid

    pltpu.sync_copy(wb_h, wbuf)
    wv = wbuf[pl.ds(0, L)]
    w0 = wv[0]
    w1 = wv[1]
    w2 = wv[2]
    w3 = wv[3]

    def body(it, carry):
        c = wid + NW * it

        @pl.when(c < NNCH)
        def _():
            base = c * NODE_CH
            pltpu.async_copy(x0h.at[pl.ds(base, NODE_CH)], x0b, semi)
            pltpu.async_copy(x1h.at[pl.ds(base, NODE_CH)], x1b, semi)
            pltpu.async_copy(x2h.at[pl.ds(base, NODE_CH)], x2b, semi)
            pltpu.async_copy(x3h.at[pl.ds(base, NODE_CH)], x3b, semi)
            pltpu.async_copy(degp_h.at[pl.ds(base, NODE_CH)], d0b, semi)
            pltpu.async_copy(degp_h.at[pl.ds(NPAD + base, NODE_CH)], d1b,
                             semi)
            for _ in range(6):
                pltpu.make_async_copy(x0h.at[pl.ds(0, NODE_CH)], x0b,
                                      semi).wait()

            def inner(v, carry2):
                sl = pl.ds(v * L, L)
                p = (x0b[sl] * w0 + x1b[sl] * w1
                     + x2b[sl] * w2 + x3b[sl] * w3)
                d = d0b[sl] + d1b[sl] + jnp.float32(1.0)
                d = jnp.maximum(d, jnp.float32(1.0))
                y = _rsqrt(d)
                nb[sl] = y
                pb[sl] = p
                zb[sl] = p * y
                return carry2

            lax.fori_loop(0, NODE_CH // L, inner, 0)
            pltpu.sync_copy(zb, z_h.at[pl.ds(base, NODE_CH)])
            pltpu.sync_copy(nb, n_h.at[pl.ds(base, NODE_CH)])
            pltpu.sync_copy(pb, p_h.at[pl.ds(base, NODE_CH)])

        return carry

    lax.fori_loop(0, (NNCH + NW - 1) // NW, body, 0)


@functools.partial(
    pl.kernel,
    out_type=jax.ShapeDtypeStruct((NC * NPAD,), f32),
    mesh=_mesh,
    compiler_params=_params,
    scratch_types=[
        pltpu.VMEM((NPAD,), f32),
        pltpu.VMEM((2, KR, RW), i32),
        pltpu.VMEM((4, KR, RW), i32),
        pltpu.VMEM((4, KR, RW), f32),
        pltpu.VMEM((DUMP // 2,), f32),
        pltpu.VMEM_SHARED((NPAD,), f32),
        pltpu.SemaphoreType.DMA,
        pltpu.SemaphoreType.DMA,
        pltpu.SemaphoreType.DMA,
        pltpu.SemaphoreType.DMA,
        pltpu.SemaphoreType.DMA,
        pltpu.SemaphoreType.DMA,
    ],
)
def _k3(src_hbm, dst_hbm, z_hbm, accp_hbm,
        ztab, sbuf, dbuf, vbuf, dmpb, shared,
        semi0, semi1, sems0, sems1, sems2, sems3):
    cid = lax.axis_index("c")
    sid = lax.axis_index("s")
    wid = sid * NC + cid
    semi = (semi0, semi1)
    sems = (sems0, sems1, sems2, sems3)

    _zero_vmem(dmpb, NODE_CH)
    _zero_shared(shared, dmpb.at[pl.ds(0, NODE_CH)], sid)
    pltpu.sync_copy(z_hbm, ztab)
    plsc.subcore_barrier()

    def start_in(i, s2, s4):
        c = wid + NW * i

        @pl.when(c < NECH)
        def _():
            pltpu.async_copy(src_hbm.at[pl.ds(c * KR, KR)], sbuf.at[s2],
                             semi[s2])
            pltpu.async_copy(dst_hbm.at[pl.ds(c * KR, KR)], dbuf.at[s4],
                             semi[s2])

    def wait_in(s2, s4):
        pltpu.make_async_copy(src_hbm.at[pl.ds(0, KR)], sbuf.at[s2],
                              semi[s2]).wait()
        pltpu.make_async_copy(dst_hbm.at[pl.ds(0, KR)], dbuf.at[s4],
                              semi[s2]).wait()

    def drain_scatter(s4):
        for j in range(KR):
            pltpu.make_async_copy(z_hbm.at[pl.ds(0, RW)], vbuf.at[s4, j],
                                  sems[s4]).wait()

    start_in(0, 0, 0)

    def body(it, carry):
        for b in range(4):
            i = 4 * it + b
            c = wid + NW * i
            start_in(i + 1, (b + 1) % 2, (b + 1) % 4)

            @pl.when(c < NECH)
            def _():
                wait_in(b % 2, b)

                @pl.when(i >= 2)
                def _():
                    drain_scatter((b + 2) % 4)

                for j in range(KR):
                    for g in range(RW // L):
                        idx = sbuf[b % 2, j, pl.ds(g * L, L)]
                        vbuf[b, j, pl.ds(g * L, L)] = plsc.load_gather(
                            ztab, [idx])
                for j in range(KR):
                    pltpu.async_copy(vbuf.at[b, j],
                                     shared.at[dbuf.at[b, j]],
                                     sems[b], add=True)

        return carry

    nit = (NECH + NW - 1) // NW
    lax.fori_loop(0, (nit + 3) // 4, body, 0)
    i_last = (NECH - 1 - wid) // NW
    for s in range(4):
        @pl.when(jnp.logical_or(i_last % 4 == s, (i_last - 1) % 4 == s))
        def _():
            drain_scatter(s)

    plsc.subcore_barrier()
    _dump_shared(shared, dmpb, accp_hbm, cid * NPAD, sid)


@functools.partial(
    pl.kernel,
    out_type=(
        jax.ShapeDtypeStruct((NW * GP,), f32),
        jax.ShapeDtypeStruct((NW * GP,), f32),
    ),
    mesh=_mesh,
    compiler_params=_params,
    scratch_types=[
        pltpu.VMEM((NODE_CH,), f32),
        pltpu.VMEM((NODE_CH,), f32),
        pltpu.VMEM((NODE_CH,), f32),
        pltpu.VMEM((NODE_CH,), f32),
        pltpu.VMEM((NODE_CH,), i32),
        pltpu.VMEM((16,), f32),
        pltpu.VMEM((GP,), f32),
        pltpu.VMEM((GP,), f32),
        pltpu.SemaphoreType.DMA,
    ],
)
def _k4(accp_h, nrm_h, p_h, batch_h, wb_h, sump_h, cntp_h,
        a0b, a1b, nb, pb, bbuf, wbuf, sumb, cntb, semi):
    cid = lax.axis_index("c")
    sid = lax.axis_index("s")
    wid = sid * NC + cid
    ones = jnp.ones((L,), f32)

    pltpu.sync_copy(wb_h, wbuf)
    bconst = wbuf[pl.ds(0, L)][4]
    _zero_vmem(sumb, GP)
    _zero_vmem(cntb, GP)

    def body(it, carry):
        c = wid + NW * it

        @pl.when(c < NNCH)
        def _():
            base = c * NODE_CH
            pltpu.async_copy(accp_h.at[pl.ds(base, NODE_CH)], a0b, semi)
            pltpu.async_copy(accp_h.at[pl.ds(NPAD + base, NODE_CH)], a1b,
                             semi)
            pltpu.async_copy(nrm_h.at[pl.ds(base, NODE_CH)], nb, semi)
            pltpu.async_copy(p_h.at[pl.ds(base, NODE_CH)], pb, semi)
            pltpu.async_copy(batch_h.at[pl.ds(base, NODE_CH)], bbuf, semi)
            for _ in range(4):
                pltpu.make_async_copy(accp_h.at[pl.ds(0, NODE_CH)], a0b,
                                      semi).wait()
            pltpu.make_async_copy(batch_h.at[pl.ds(0, NODE_CH)], bbuf,
                                  semi).wait()

            def inner(v, carry2):
                sl = pl.ds(v * L, L)
                y = nb[sl]
                o = y * (a0b[sl] + a1b[sl]) + pb[sl] * y * y + bconst
                bi = bbuf[sl]
                plsc.addupdate_scatter(sumb, [bi], o)
                plsc.addupdate_scatter(cntb, [bi], ones)
                return carry2

            lax.fori_loop(0, NODE_CH // L, inner, 0)

        return carry

    lax.fori_loop(0, (NNCH + NW - 1) // NW, body, 0)
    pltpu.sync_copy(sumb, sump_h.at[pl.ds(wid * GP, GP)])
    pltpu.sync_copy(cntb, cntp_h.at[pl.ds(wid * GP, GP)])


@functools.partial(
    pl.kernel,
    out_type=jax.ShapeDtypeStruct((G,), f32),
    mesh=_mesh,
    compiler_params=_params,
    scratch_types=[
        pltpu.VMEM((NW * GP,), f32),
        pltpu.VMEM((NW * GP,), f32),
        pltpu.VMEM((2 * L,), f32),
    ],
)
def _k5(sump_h, cntp_h, logits_h, sbuf, cbuf, obuf):
    cid = lax.axis_index("c")
    sid = lax.axis_index("s")
    wid = sid * NC + cid
    per_w = G // NW

    pltpu.sync_copy(sump_h, sbuf)
    pltpu.sync_copy(cntp_h, cbuf)
    base = wid * per_w
    for v in range(per_w // L):
        s = jnp.zeros((L,), f32)
        cnt = jnp.zeros((L,), f32)
        for w in range(NW):
            s = s + sbuf[pl.ds(w * GP + base + v * L, L)]
            cnt = cnt + cbuf[pl.ds(w * GP + base + v * L, L)]
        obuf[pl.ds(v * L, L)] = s / jnp.maximum(cnt, jnp.float32(1.0))
    pltpu.sync_copy(obuf, logits_h.at[pl.ds(base, per_w)])


def kernel(x, edge_index, batch, W, b):
    src2 = edge_index[0].reshape(EROWS, RW)
    dst2 = edge_index[1].reshape(EROWS, RW)
    xp = jnp.pad(x, ((0, NPAD - N), (0, 0)))
    x0, x1, x2, x3 = (xp[:, j] for j in range(4))
    batchp = jnp.pad(batch, (0, NPAD - N), constant_values=G)
    wb = jnp.zeros((16,), f32).at[:4].set(W[:, 0]).at[4].set(b[0])

    degp = _k1(dst2)
    z, nrm, p = _k2(x0, x1, x2, x3, degp, wb)
    accp = _k3(src2, dst2, z)
    sump, cntp = _k4(accp, nrm, p, batchp, wb)
    return _k5(sump, cntp)


# speedup vs baseline: 495.1289x; 1.0037x over previous
import functools

import jax
import jax.numpy as jnp
from jax import lax
from jax.experimental import pallas as pl
from jax.experimental.pallas import tpu as pltpu
from jax.experimental.pallas import tpu_sc as plsc

N = 100000
E = 6400000
G = 1024

NC = 2
NS = 16
NW = NC * NS
L = 16

RW = 128
KR = 16
ECH = KR * RW
NECH = E // ECH
EROWS = E // RW

NODE_CH = 1024
NPAD = 100352
NNCH = NPAD // NODE_CH
DUMP = NPAD // NS
BLK = NW * NODE_CH

GP = 1056

_mesh = plsc.VectorSubcoreMesh(
    core_axis_name="c", subcore_axis_name="s", num_cores=NC, num_subcores=NS)
_params = pltpu.CompilerParams(needs_layout_passes=False)
f32 = jnp.float32
i32 = jnp.int32


def _rsqrt(d):
    i = lax.bitcast_convert_type(d, i32)
    i = jnp.int32(0x5F3759DF) - lax.shift_right_logical(i, 1)
    y = lax.bitcast_convert_type(i, f32)
    for _ in range(3):
        y = y * (jnp.float32(1.5) - jnp.float32(0.5) * d * y * y)
    return y


def _zero_vmem(ref, n):
    for v in range(n // L):
        ref[pl.ds(v * L, L)] = jnp.zeros((L,), f32)


def _zero_vmem_big(ref, n):
    def body(it, carry):
        base = it * 256
        for k in range(16):
            ref[pl.ds(base + k * L, L)] = jnp.zeros((L,), f32)
        return carry

    lax.fori_loop(0, n // 256, body, 0)


def _zero_shared(shared, zbuf, sid):
    nz = shared.shape[0] // NODE_CH
    for it in range((nz + NS - 1) // NS):
        c = sid + NS * it

        @pl.when(c < nz)
        def _():
            pltpu.sync_copy(zbuf, shared.at[pl.ds(c * NODE_CH, NODE_CH)])


def _dump_shared(shared, dbuf, hbm, base, sid):
    half = DUMP // 2
    for k in range(2):
        off = sid * DUMP + k * half
        pltpu.sync_copy(shared.at[pl.ds(off, half)], dbuf)
        pltpu.sync_copy(dbuf, hbm.at[pl.ds(base + off, half)])


@functools.partial(
    pl.kernel,
    out_type=jax.ShapeDtypeStruct((NC * NPAD,), f32),
    mesh=_mesh,
    compiler_params=_params,
    scratch_types=[
        pltpu.VMEM((4, KR, RW), i32),
        pltpu.VMEM((RW,), f32),
        pltpu.VMEM((DUMP // 2,), f32),
        pltpu.VMEM_SHARED((NPAD,), f32),
        pltpu.SemaphoreType.DMA,
        pltpu.SemaphoreType.DMA,
        pltpu.SemaphoreType.DMA,
        pltpu.SemaphoreType.DMA,
        pltpu.SemaphoreType.DMA,
        pltpu.SemaphoreType.DMA,
    ],
)
def _k1(dst_hbm, degp_hbm, dbuf, ones_v, dmpb, shared,
        semi0, semi1, sems0, sems1, sems2, sems3):
    cid = lax.axis_index("c")
    sid = lax.axis_index("s")
    wid = sid * NC + cid
    semi = (semi0, semi1)
    sems = (sems0, sems1, sems2, sems3)

    _zero_vmem(dmpb, NODE_CH)
    for v in range(RW // L):
        ones_v[pl.ds(v * L, L)] = jnp.ones((L,), f32)
    _zero_shared(shared, dmpb.at[pl.ds(0, NODE_CH)], sid)
    plsc.subcore_barrier()

    def start_in(i, s4):
        c = wid + NW * i

        @pl.when(c < NECH)
        def _():
            pltpu.async_copy(dst_hbm.at[pl.ds(c * KR, KR)], dbuf.at[s4],
                             semi[s4 % 2])

    def wait_in(s4):
        pltpu.make_async_copy(dst_hbm.at[pl.ds(0, KR)], dbuf.at[s4],
                              semi[s4 % 2]).wait()

    def drain_scatter(s4):
        for j in range(KR):
            pltpu.make_async_copy(degp_hbm.at[pl.ds(0, RW)], ones_v,
                                  sems[s4]).wait()

    start_in(0, 0)

    def body(it, carry):
        for b in range(4):
            i = 4 * it + b
            c = wid + NW * i
            start_in(i + 1, (b + 1) % 4)

            @pl.when(c < NECH)
            def _():
                wait_in(b)

                @pl.when(i >= 2)
                def _():
                    drain_scatter((b + 2) % 4)

                for j in range(KR):
                    pltpu.async_copy(ones_v, shared.at[dbuf.at[b, j]],
                                     sems[b], add=True)

        return carry

    nit = (NECH + NW - 1) // NW
    lax.fori_loop(0, (nit + 3) // 4, body, 0)
    i_last = (NECH - 1 - wid) // NW
    for s in range(4):
        @pl.when(jnp.logical_or(i_last % 4 == s, (i_last - 1) % 4 == s))
        def _():
            drain_scatter(s)

    plsc.subcore_barrier()
    _dump_shared(shared, dmpb, degp_hbm, cid * NPAD, sid)


@functools.partial(
    pl.kernel,
    out_type=(
        jax.ShapeDtypeStruct((NPAD,), f32),
        jax.ShapeDtypeStruct((NPAD,), f32),
        jax.ShapeDtypeStruct((NPAD,), f32),
    ),
    mesh=_mesh,
    compiler_params=_params,
    scratch_types=[
        pltpu.VMEM((NODE_CH,), f32),
        pltpu.VMEM((NODE_CH,), f32),
        pltpu.VMEM((NODE_CH,), f32),
        pltpu.VMEM((NODE_CH,), f32),
        pltpu.VMEM((NODE_CH,), f32),
        pltpu.VMEM((NODE_CH,), f32),
        pltpu.VMEM((16,), f32),
        pltpu.VMEM((NODE_CH,), f32),
        pltpu.VMEM((NODE_CH,), f32),
        pltpu.VMEM((NODE_CH,), f32),
        pltpu.SemaphoreType.DMA,
    ],
)
def _k2(x0h, x1h, x2h, x3h, degp_h, wb_h, z_h, n_h, p_h,
        x0b, x1b, x2b, x3b, d0b, d1b, wbuf, zb, nb, pb, semi):
    cid = lax.axis_index("c")
    sid = lax.axis_index("s")
    wid = sid * NC + cid

    pltpu.sync_copy(wb_h, wbuf)
    wv = wbuf[pl.ds(0, L)]
    w0 = wv[0]
    w1 = wv[1]
    w2 = wv[2]
    w3 = wv[3]

    def body(it, carry):
        c = wid + NW * it

        @pl.when(c < NNCH)
        def _():
            base = c * NODE_CH
            pltpu.async_copy(x0h.at[pl.ds(base, NODE_CH)], x0b, semi)
            pltpu.async_copy(x1h.at[pl.ds(base, NODE_CH)], x1b, semi)
            pltpu.async_copy(x2h.at[pl.ds(base, NODE_CH)], x2b, semi)
            pltpu.async_copy(x3h.at[pl.ds(base, NODE_CH)], x3b, semi)
            pltpu.async_copy(degp_h.at[pl.ds(base, NODE_CH)], d0b, semi)
            pltpu.async_copy(degp_h.at[pl.ds(NPAD + base, NODE_CH)], d1b,
                             semi)
            for _ in range(6):
                pltpu.make_async_copy(x0h.at[pl.ds(0, NODE_CH)], x0b,
                                      semi).wait()

            def inner(v, carry2):
                sl = pl.ds(v * L, L)
                p = (x0b[sl] * w0 + x1b[sl] * w1
                     + x2b[sl] * w2 + x3b[sl] * w3)
                d = d0b[sl] + d1b[sl] + jnp.float32(1.0)
                d = jnp.maximum(d, jnp.float32(1.0))
                y = _rsqrt(d)
                nb[sl] = y
                pb[sl] = p
                zb[sl] = p * y
                return carry2

            lax.fori_loop(0, NODE_CH // L, inner, 0)
            pltpu.sync_copy(zb, z_h.at[pl.ds(base, NODE_CH)])
            pltpu.sync_copy(nb, n_h.at[pl.ds(base, NODE_CH)])
            pltpu.sync_copy(pb, p_h.at[pl.ds(base, NODE_CH)])

        return carry

    lax.fori_loop(0, (NNCH + NW - 1) // NW, body, 0)


@functools.partial(
    pl.kernel,
    out_type=jax.ShapeDtypeStruct((NC * NPAD,), f32),
    mesh=_mesh,
    compiler_params=_params,
    scratch_types=[
        pltpu.VMEM((NPAD,), f32),
        pltpu.VMEM((2, KR, RW), i32),
        pltpu.VMEM((4, KR, RW), i32),
        pltpu.VMEM((4, KR, RW), f32),
        pltpu.VMEM((DUMP // 2,), f32),
        pltpu.VMEM_SHARED((NPAD,), f32),
        pltpu.SemaphoreType.DMA,
        pltpu.SemaphoreType.DMA,
        pltpu.SemaphoreType.DMA,
        pltpu.SemaphoreType.DMA,
        pltpu.SemaphoreType.DMA,
        pltpu.SemaphoreType.DMA,
    ],
)
def _k3(src_hbm, dst_hbm, z_hbm, accp_hbm,
        ztab, sbuf, dbuf, vbuf, dmpb, shared,
        semi0, semi1, sems0, sems1, sems2, sems3):
    cid = lax.axis_index("c")
    sid = lax.axis_index("s")
    wid = sid * NC + cid
    semi = (semi0, semi1)
    sems = (sems0, sems1, sems2, sems3)

    _zero_vmem(dmpb, NODE_CH)
    _zero_shared(shared, dmpb.at[pl.ds(0, NODE_CH)], sid)
    pltpu.sync_copy(z_hbm, ztab)
    plsc.subcore_barrier()

    def start_in(i, s2, s4):
        c = wid + NW * i

        @pl.when(c < NECH)
        def _():
            pltpu.async_copy(src_hbm.at[pl.ds(c * KR, KR)], sbuf.at[s2],
                             semi[s2])
            pltpu.async_copy(dst_hbm.at[pl.ds(c * KR, KR)], dbuf.at[s4],
                             semi[s2])

    def wait_in(s2, s4):
        pltpu.make_async_copy(src_hbm.at[pl.ds(0, KR)], sbuf.at[s2],
                              semi[s2]).wait()
        pltpu.make_async_copy(dst_hbm.at[pl.ds(0, KR)], dbuf.at[s4],
                              semi[s2]).wait()

    def drain_scatter(s4):
        for j in range(KR):
            pltpu.make_async_copy(z_hbm.at[pl.ds(0, RW)], vbuf.at[s4, j],
                                  sems[s4]).wait()

    start_in(0, 0, 0)

    def body(it, carry):
        for b in range(4):
            i = 4 * it + b
            c = wid + NW * i
            start_in(i + 1, (b + 1) % 2, (b + 1) % 4)

            @pl.when(c < NECH)
            def _():
                wait_in(b % 2, b)
                for j in range(KR):
                    for g in range(RW // L):
                        idx = sbuf[b % 2, j, pl.ds(g * L, L)]
                        vbuf[b, j, pl.ds(g * L, L)] = plsc.load_gather(
                            ztab, [idx])

                @pl.when(i >= 2)
                def _():
                    drain_scatter((b + 2) % 4)

                for j in range(KR):
                    pltpu.async_copy(vbuf.at[b, j],
                                     shared.at[dbuf.at[b, j]],
                                     sems[b], add=True)

        return carry

    nit = (NECH + NW - 1) // NW
    lax.fori_loop(0, (nit + 3) // 4, body, 0)
    i_last = (NECH - 1 - wid) // NW
    for s in range(4):
        @pl.when(jnp.logical_or(i_last % 4 == s, (i_last - 1) % 4 == s))
        def _():
            drain_scatter(s)

    plsc.subcore_barrier()
    _dump_shared(shared, dmpb, accp_hbm, cid * NPAD, sid)


@functools.partial(
    pl.kernel,
    out_type=(
        jax.ShapeDtypeStruct((NW * GP,), f32),
        jax.ShapeDtypeStruct((NW * GP,), f32),
    ),
    mesh=_mesh,
    compiler_params=_params,
    scratch_types=[
        pltpu.VMEM((NODE_CH,), f32),
        pltpu.VMEM((NODE_CH,), f32),
        pltpu.VMEM((NODE_CH,), f32),
        pltpu.VMEM((NODE_CH,), f32),
        pltpu.VMEM((NODE_CH,), i32),
        pltpu.VMEM((16,), f32),
        pltpu.VMEM((GP,), f32),
        pltpu.VMEM((GP,), f32),
        pltpu.SemaphoreType.DMA,
    ],
)
def _k4(accp_h, nrm_h, p_h, batch_h, wb_h, sump_h, cntp_h,
        a0b, a1b, nb, pb, bbuf, wbuf, sumb, cntb, semi):
    cid = lax.axis_index("c")
    sid = lax.axis_index("s")
    wid = sid * NC + cid
    ones = jnp.ones((L,), f32)

    pltpu.sync_copy(wb_h, wbuf)
    bconst = wbuf[pl.ds(0, L)][4]
    _zero_vmem(sumb, GP)
    _zero_vmem(cntb, GP)

    def body(it, carry):
        c = wid + NW * it

        @pl.when(c < NNCH)
        def _():
            base = c * NODE_CH
            pltpu.async_copy(accp_h.at[pl.ds(base, NODE_CH)], a0b, semi)
            pltpu.async_copy(accp_h.at[pl.ds(NPAD + base, NODE_CH)], a1b,
                             semi)
            pltpu.async_copy(nrm_h.at[pl.ds(base, NODE_CH)], nb, semi)
            pltpu.async_copy(p_h.at[pl.ds(base, NODE_CH)], pb, semi)
            pltpu.async_copy(batch_h.at[pl.ds(base, NODE_CH)], bbuf, semi)
            for _ in range(4):
                pltpu.make_async_copy(accp_h.at[pl.ds(0, NODE_CH)], a0b,
                                      semi).wait()
            pltpu.make_async_copy(batch_h.at[pl.ds(0, NODE_CH)], bbuf,
                                  semi).wait()

            def inner(v, carry2):
                sl = pl.ds(v * L, L)
                y = nb[sl]
                o = y * (a0b[sl] + a1b[sl]) + pb[sl] * y * y + bconst
                bi = bbuf[sl]
                plsc.addupdate_scatter(sumb, [bi], o)
                plsc.addupdate_scatter(cntb, [bi], ones)
                return carry2

            lax.fori_loop(0, NODE_CH // L, inner, 0)

        return carry

    lax.fori_loop(0, (NNCH + NW - 1) // NW, body, 0)
    pltpu.sync_copy(sumb, sump_h.at[pl.ds(wid * GP, GP)])
    pltpu.sync_copy(cntb, cntp_h.at[pl.ds(wid * GP, GP)])


@functools.partial(
    pl.kernel,
    out_type=jax.ShapeDtypeStruct((G,), f32),
    mesh=_mesh,
    compiler_params=_params,
    scratch_types=[
        pltpu.VMEM((NW * GP,), f32),
        pltpu.VMEM((NW * GP,), f32),
        pltpu.VMEM((2 * L,), f32),
    ],
)
def _k5(sump_h, cntp_h, logits_h, sbuf, cbuf, obuf):
    cid = lax.axis_index("c")
    sid = lax.axis_index("s")
    wid = sid * NC + cid
    per_w = G // NW

    pltpu.sync_copy(sump_h, sbuf)
    pltpu.sync_copy(cntp_h, cbuf)
    base = wid * per_w
    for v in range(per_w // L):
        s = jnp.zeros((L,), f32)
        cnt = jnp.zeros((L,), f32)
        for w in range(NW):
            s = s + sbuf[pl.ds(w * GP + base + v * L, L)]
            cnt = cnt + cbuf[pl.ds(w * GP + base + v * L, L)]
        obuf[pl.ds(v * L, L)] = s / jnp.maximum(cnt, jnp.float32(1.0))
    pltpu.sync_copy(obuf, logits_h.at[pl.ds(base, per_w)])


def kernel(x, edge_index, batch, W, b):
    src2 = edge_index[0].reshape(EROWS, RW)
    dst2 = edge_index[1].reshape(EROWS, RW)
    xp = jnp.pad(x, ((0, NPAD - N), (0, 0)))
    x0, x1, x2, x3 = (xp[:, j] for j in range(4))
    batchp = jnp.pad(batch, (0, NPAD - N), constant_values=G)
    wb = jnp.zeros((16,), f32).at[:4].set(W[:, 0]).at[4].set(b[0])

    degp = _k1(dst2)
    z, nrm, p = _k2(x0, x1, x2, x3, degp, wb)
    accp = _k3(src2, dst2, z)
    sump, cntp = _k4(accp, nrm, p, batchp, wb)
    return _k5(sump, cntp)
